# baseline (speedup 1.0000x reference)
import functools

import jax
import jax.numpy as jnp
from jax.experimental import pallas as pl
from jax.experimental.pallas import tpu as pltpu

B = 16384
D = 64
L = 1024
K = 8
BM = 512


_NET = [(0, 1), (2, 3), (4, 5), (6, 7),
        (0, 2), (1, 3), (4, 6), (5, 7),
        (1, 2), (5, 6), (0, 4), (3, 7),
        (1, 5), (2, 6),
        (1, 4), (3, 6),
        (2, 4), (3, 5),
        (3, 4)]


def _fused_body(x_ref, b_pre_ref, W_enc_ref, b_enc_ref, W_dec_ref,
                acts_ref, recon_ref, pre_ref):
    x = x_ref[...]
    b_pre = b_pre_ref[...]
    W_enc = W_enc_ref[...]
    b_enc = b_enc_ref[...]
    W_dec = W_dec_ref[...]

    pre_ref[...] = jnp.dot(x - b_pre, W_enc,
                           preferred_element_type=jnp.float32) + b_enc

    NC = L // 128
    GM = 32
    for gi in range(BM // GM):
        g = pre_ref[pl.ds(gi * GM, GM), :]
        S = [g[:, c * 128:(c + 1) * 128] for c in range(NC)]
        for a, b in _NET:
            hi = jnp.maximum(S[a], S[b])
            lo = jnp.minimum(S[a], S[b])
            S[a], S[b] = hi, lo
        for r in range(K - 1):
            t = jnp.max(S[0], axis=1, keepdims=True)
            pop = S[0] == t
            for a in range(K - 1 - r):
                S[a] = jnp.where(pop, S[a + 1], S[a])
        t = jnp.max(S[0], axis=1, keepdims=True)
        t = jnp.maximum(t, jnp.float32(1.1754944e-38))
        acts_ref[pl.ds(gi * GM, GM), :] = jnp.where(
            g >= t, g, jnp.float32(0.0))

    recon_ref[...] = jnp.dot(acts_ref[...].astype(jnp.bfloat16),
                             W_dec.astype(jnp.bfloat16),
                             preferred_element_type=jnp.float32) + b_pre


@jax.jit
def kernel(x, b_pre, W_enc, b_enc, W_dec):
    grid = (B // BM,)
    acts, recon = pl.pallas_call(
        _fused_body,
        grid=grid,
        in_specs=[
            pl.BlockSpec((BM, D), lambda i: (i, 0)),
            pl.BlockSpec((1, D), lambda i: (0, 0)),
            pl.BlockSpec((D, L), lambda i: (0, 0)),
            pl.BlockSpec((1, L), lambda i: (0, 0)),
            pl.BlockSpec((L, D), lambda i: (0, 0)),
        ],
        out_specs=[
            pl.BlockSpec((BM, L), lambda i: (i, 0)),
            pl.BlockSpec((BM, D), lambda i: (i, 0)),
        ],
        out_shape=[
            jax.ShapeDtypeStruct((B, L), jnp.float32),
            jax.ShapeDtypeStruct((B, D), jnp.float32),
        ],
        scratch_shapes=[pltpu.VMEM((BM, L), jnp.float32)],
        compiler_params=pltpu.CompilerParams(
            dimension_semantics=("arbitrary",),
        ),
    )(x, b_pre.reshape(1, D), W_enc, b_enc.reshape(1, L), W_dec)
    return acts, recon


# speedup vs baseline: 1.0009x; 1.0009x over previous
import functools

import jax
import jax.numpy as jnp
from jax.experimental import pallas as pl
from jax.experimental.pallas import tpu as pltpu

B = 16384
D = 64
L = 1024
K = 8
BM = 512


_NET = [(0, 1), (2, 3), (4, 5), (6, 7),
        (0, 2), (1, 3), (4, 6), (5, 7),
        (1, 2), (5, 6), (0, 4), (3, 7),
        (1, 5), (2, 6),
        (1, 4), (3, 6),
        (2, 4), (3, 5),
        (3, 4)]


def _fused_body(x_ref, b_pre_ref, W_enc_ref, b_enc_ref, W_dec_ref,
                acts_ref, recon_ref):
    x = x_ref[...]
    b_pre = b_pre_ref[...]
    W_enc = W_enc_ref[...]
    b_enc = b_enc_ref[...]
    W_dec = W_dec_ref[...]

    pre = jnp.dot(x - b_pre, W_enc,
                  preferred_element_type=jnp.float32) + b_enc

    NC = L // 128
    S = [pre[:, c * 128:(c + 1) * 128] for c in range(NC)]
    for a, b in _NET:
        hi = jnp.maximum(S[a], S[b])
        lo = jnp.minimum(S[a], S[b])
        S[a], S[b] = hi, lo
    for r in range(K - 1):
        t = jnp.max(S[0], axis=1, keepdims=True)
        pop = S[0] == t
        for a in range(K - 1 - r):
            S[a] = jnp.where(pop, S[a + 1], S[a])
    t = jnp.max(S[0], axis=1, keepdims=True)

    t = jnp.maximum(t, jnp.float32(1.1754944e-38))
    acts = jnp.where(pre >= t, pre, jnp.float32(0.0))
    acts_ref[...] = acts

    recon_ref[...] = jnp.dot(acts, W_dec,
                             preferred_element_type=jnp.float32) + b_pre


@jax.jit
def kernel(x, b_pre, W_enc, b_enc, W_dec):
    grid = (B // BM,)
    acts, recon = pl.pallas_call(
        _fused_body,
        grid=grid,
        in_specs=[
            pl.BlockSpec((BM, D), lambda i: (i, 0)),
            pl.BlockSpec((1, D), lambda i: (0, 0)),
            pl.BlockSpec((D, L), lambda i: (0, 0)),
            pl.BlockSpec((1, L), lambda i: (0, 0)),
            pl.BlockSpec((L, D), lambda i: (0, 0)),
        ],
        out_specs=[
            pl.BlockSpec((BM, L), lambda i: (i, 0)),
            pl.BlockSpec((BM, D), lambda i: (i, 0)),
        ],
        out_shape=[
            jax.ShapeDtypeStruct((B, L), jnp.float32),
            jax.ShapeDtypeStruct((B, D), jnp.float32),
        ],

        compiler_params=pltpu.CompilerParams(
            dimension_semantics=("parallel",),
        ),
    )(x, b_pre.reshape(1, D), W_enc, b_enc.reshape(1, L), W_dec)
    return acts, recon


# speedup vs baseline: 1.0682x; 1.0672x over previous
import jax
import jax.numpy as jnp
from jax.experimental import pallas as pl
from jax.experimental.pallas import tpu as pltpu

B = 16384
D = 64
L = 1024
K = 8
BM = 512
NB = B // BM
NP = NB // 2
PAIR = 2 * BM

_NET = [(0, 1), (2, 3), (4, 5), (6, 7),
        (0, 2), (1, 3), (4, 6), (5, 7),
        (1, 2), (5, 6), (0, 4), (3, 7),
        (1, 5), (2, 6),
        (1, 4), (3, 6),
        (2, 4), (3, 5),
        (3, 4)]


def _select(pre):
    NC = L // 128
    S = [pre[:, c * 128:(c + 1) * 128] for c in range(NC)]
    for a, b in _NET:
        hi = jnp.maximum(S[a], S[b])
        lo = jnp.minimum(S[a], S[b])
        S[a], S[b] = hi, lo
    for r in range(K - 1):
        t = jnp.max(S[0], axis=1, keepdims=True)
        pop = S[0] == t
        for a in range(K - 1 - r):
            S[a] = jnp.where(pop, S[a + 1], S[a])
    t = jnp.max(S[0], axis=1, keepdims=True)
    t = jnp.maximum(t, jnp.float32(1.1754944e-38))
    return jnp.where(pre >= t, pre, jnp.float32(0.0))


def _fused_body(x_ref, b_pre_ref, W_enc_ref, b_enc_ref, W_dec_ref,
                acts_ref, recon_ref, pA, pB):
    x = x_ref[...]
    b_pre = b_pre_ref[...]
    W_enc = W_enc_ref[...]
    b_enc = b_enc_ref[...]
    W_dec = W_dec_ref[...]

    for half, pref in ((0, pA), (1, pB)):
        acts_h = _select(pref[...])
        acts_ref[pl.ds(half * BM, BM), :] = acts_h
        recon_ref[pl.ds(half * BM, BM), :] = jnp.dot(
            acts_h, W_dec, preferred_element_type=jnp.float32) + b_pre

    xc = x - b_pre
    pA[...] = jnp.dot(xc[:BM], W_enc,
                      preferred_element_type=jnp.float32) + b_enc
    pB[...] = jnp.dot(xc[BM:], W_enc,
                      preferred_element_type=jnp.float32) + b_enc


@jax.jit
def kernel(x, b_pre, W_enc, b_enc, W_dec):
    grid = (NP + 1,)
    acts, recon = pl.pallas_call(
        _fused_body,
        grid=grid,
        in_specs=[
            pl.BlockSpec((PAIR, D), lambda j: (jnp.minimum(j, NP - 1), 0)),
            pl.BlockSpec((1, D), lambda j: (0, 0)),
            pl.BlockSpec((D, L), lambda j: (0, 0)),
            pl.BlockSpec((1, L), lambda j: (0, 0)),
            pl.BlockSpec((L, D), lambda j: (0, 0)),
        ],
        out_specs=[
            pl.BlockSpec((PAIR, L), lambda j: (jnp.maximum(j - 1, 0), 0)),
            pl.BlockSpec((PAIR, D), lambda j: (jnp.maximum(j - 1, 0), 0)),
        ],
        out_shape=[
            jax.ShapeDtypeStruct((B, L), jnp.float32),
            jax.ShapeDtypeStruct((B, D), jnp.float32),
        ],
        scratch_shapes=[
            pltpu.VMEM((BM, L), jnp.float32),
            pltpu.VMEM((BM, L), jnp.float32),
        ],
        compiler_params=pltpu.CompilerParams(
            dimension_semantics=("arbitrary",),
        ),
    )(x, b_pre.reshape(1, D), W_enc, b_enc.reshape(1, L), W_dec)
    return acts, recon
